# baseline (speedup 1.0000x reference)
import jax
import jax.numpy as jnp
from jax import lax
from jax.experimental import pallas as pl
from jax.experimental.pallas import tpu as pltpu
from jax.experimental.pallas import tpu_sc as plsc

_B = 16384
_D = 32
_NC = 2
_NS = 16
_NW = _NC * _NS
_BPW = _B // _NW
_CHUNK = 128
_NCHUNK = _BPW // _CHUNK
_GB = 4


def _sc_body(user_ref, item_ref, ut_ref, it_ref, par_ref, out_ref,
             idx_u, idx_i, cols_u, cols_i, out_v, w_b, sem_u, sem_i):
    wid = lax.axis_index("s") * _NC + lax.axis_index("c")
    pltpu.sync_copy(user_ref.at[wid], idx_u)
    pltpu.sync_copy(item_ref.at[wid], idx_i)
    pltpu.sync_copy(par_ref, w_b)
    for c in range(_D):
        for j in range(_NCHUNK):
            sl = pl.ds(j * _CHUNK, _CHUNK)
            pltpu.async_copy(ut_ref.at[c].at[idx_u.at[j]], cols_u.at[c, sl], sem_u)
            pltpu.async_copy(it_ref.at[c].at[idx_i.at[j]], cols_i.at[c, sl], sem_i)
    pltpu.make_async_copy(ut_ref.at[pl.ds(0, _D), pl.ds(0, _BPW)], cols_u, sem_u).wait()
    pltpu.make_async_copy(it_ref.at[pl.ds(0, _D), pl.ds(0, _BPW)], cols_i, sem_i).wait()

    def step(it, carry):
        for g in range(_GB):
            base = (it * _GB + g) * 16
            b16 = pl.ds(base, 16)
            acc = w_b[2 * _D]
            for c in range(_D):
                acc = acc + cols_u[c, b16] * w_b[c] + cols_i[c, b16] * w_b[_D + c]
            out_v[b16] = 1.0 / (1.0 + jnp.exp(-acc))
        return carry

    lax.fori_loop(0, _BPW // (16 * _GB), step, 0)
    pltpu.sync_copy(out_v, out_ref.at[pl.ds(wid * _BPW, _BPW)])


_sc_call = pl.kernel(
    _sc_body,
    out_type=jax.ShapeDtypeStruct((_B,), jnp.float32),
    mesh=plsc.VectorSubcoreMesh(core_axis_name="c", subcore_axis_name="s"),
    scratch_types=[
        pltpu.VMEM((_NCHUNK, _CHUNK), jnp.int32),
        pltpu.VMEM((_NCHUNK, _CHUNK), jnp.int32),
        pltpu.VMEM((_D, _BPW), jnp.float32),
        pltpu.VMEM((_D, _BPW), jnp.float32),
        pltpu.VMEM((_BPW,), jnp.float32),
        pltpu.VMEM((2 * _D + 1, 16), jnp.float32),
        pltpu.SemaphoreType.DMA,
        pltpu.SemaphoreType.DMA,
    ],
    compiler_params=pltpu.CompilerParams(
        needs_layout_passes=False, use_tc_tiling_on_sc=False),
)


def kernel(user, item, user_table, item_table, fc_w, fc_b):
    u3 = user.reshape(_NW, _NCHUNK, _CHUNK)
    i3 = item.reshape(_NW, _NCHUNK, _CHUNK)
    params = jnp.concatenate([fc_w.reshape(-1), fc_b.reshape(1)])
    params_b = jnp.broadcast_to(params[:, None], (2 * _D + 1, 16))
    out = _sc_call(u3, i3, user_table.T, item_table.T, params_b)
    return out.reshape(_B, 1)


# speedup vs baseline: 21.4567x; 21.4567x over previous
import jax
import jax.numpy as jnp
from jax import lax
from jax.experimental import pallas as pl
from jax.experimental.pallas import tpu as pltpu
from jax.experimental.pallas import tpu_sc as plsc

_B = 16384
_D = 32
_NC = 2
_NS = 16
_NW = _NC * _NS
_BPW = _B // _NW
_CW = 4
_NCH = _BPW // _CW


def _sc_body(user_ref, item_ref, ut_ref, it_ref, par_ref, out_ref,
             idx_u, idx_i, fb_u, fb_i, out_v, w_b,
             sem_u0, sem_u1, sem_i0, sem_i1):
    wid = lax.axis_index("s") * _NC + lax.axis_index("c")
    pltpu.sync_copy(user_ref.at[wid], idx_u.at[pl.ds(0, _BPW)])
    pltpu.sync_copy(item_ref.at[wid], idx_i.at[pl.ds(0, _BPW)])
    pltpu.sync_copy(par_ref, w_b)

    def start_chunk(c, bsel, sem_u, sem_i):
        iu = idx_u[pl.ds(c * _CW, 16)]
        ii = idx_i[pl.ds(c * _CW, 16)]
        for r in range(_CW):
            su = pl.multiple_of((iu[r] // 128) * 128, 128)
            si = pl.multiple_of((ii[r] // 128) * 128, 128)
            pltpu.async_copy(ut_ref.at[:, pl.ds(su, 128)], fb_u.at[bsel, r], sem_u)
            pltpu.async_copy(it_ref.at[:, pl.ds(si, 128)], fb_i.at[bsel, r], sem_i)

    def drain(bsel, sem_u, sem_i):
        pltpu.make_async_copy(ut_ref.at[:, pl.ds(0, _CW * 128)],
                              fb_u.at[bsel], sem_u).wait()
        pltpu.make_async_copy(it_ref.at[:, pl.ds(0, _CW * 128)],
                              fb_i.at[bsel], sem_i).wait()

    wu0 = w_b[0]
    wu1 = w_b[1]
    wi0 = w_b[2]
    wi1 = w_b[3]
    bias = w_b[4]
    lane = jnp.arange(16, dtype=jnp.int32)
    ci = jnp.arange(16, dtype=jnp.int32)

    def compute_chunk(c, bsel, acc):
        acc = jnp.where((c % 4) == 0, bias, acc)
        iu = idx_u[pl.ds(c * _CW, 16)]
        ii = idx_i[pl.ds(c * _CW, 16)]
        for r in range(_CW):
            rlu = jnp.full((16,), iu[r] % 128, jnp.int32)
            rli = jnp.full((16,), ii[r] % 128, jnp.int32)
            gu0 = plsc.load_gather(fb_u.at[bsel, r], [ci, rlu])
            gu1 = plsc.load_gather(fb_u.at[bsel, r], [ci + 16, rlu])
            gi0 = plsc.load_gather(fb_i.at[bsel, r], [ci, rli])
            gi1 = plsc.load_gather(fb_i.at[bsel, r], [ci + 16, rli])
            t = gu0 * wu0 + gu1 * wu1 + gi0 * wi0 + gi1 * wi1
            acc = jnp.where(lane == (c % 4) * _CW + r, acc + jnp.sum(t), acc)

        @pl.when((c % 4) == 3)
        def _():
            out_v[pl.ds((c // 4) * 16, 16)] = 1.0 / (1.0 + jnp.exp(-acc))
        return acc

    start_chunk(0, 0, sem_u0, sem_i0)

    def step(k, acc):
        c0 = 2 * k
        start_chunk(c0 + 1, 1, sem_u1, sem_i1)
        drain(0, sem_u0, sem_i0)
        acc = compute_chunk(c0, 0, acc)

        @pl.when(c0 + 2 < _NCH)
        def _():
            start_chunk(c0 + 2, 0, sem_u0, sem_i0)
        drain(1, sem_u1, sem_i1)
        acc = compute_chunk(c0 + 1, 1, acc)
        return acc

    lax.fori_loop(0, _NCH // 2, step, jnp.zeros((16,), jnp.float32))
    pltpu.sync_copy(out_v, out_ref.at[pl.ds(wid * _BPW, _BPW)])


_sc_call = pl.kernel(
    _sc_body,
    out_type=jax.ShapeDtypeStruct((_B,), jnp.float32),
    mesh=plsc.VectorSubcoreMesh(core_axis_name="c", subcore_axis_name="s"),
    scratch_types=[
        pltpu.VMEM((_BPW + 16,), jnp.int32),
        pltpu.VMEM((_BPW + 16,), jnp.int32),
        pltpu.VMEM((2, _CW, _D, 128), jnp.float32),
        pltpu.VMEM((2, _CW, _D, 128), jnp.float32),
        pltpu.VMEM((_BPW,), jnp.float32),
        pltpu.VMEM((5, 16), jnp.float32),
        pltpu.SemaphoreType.DMA,
        pltpu.SemaphoreType.DMA,
        pltpu.SemaphoreType.DMA,
        pltpu.SemaphoreType.DMA,
    ],
    compiler_params=pltpu.CompilerParams(
        needs_layout_passes=False, use_tc_tiling_on_sc=True),
)


def kernel(user, item, user_table, item_table, fc_w, fc_b):
    u2 = user.reshape(_NW, _BPW)
    i2 = item.reshape(_NW, _BPW)
    w = fc_w.reshape(4, 16)
    params = jnp.concatenate(
        [w, jnp.broadcast_to(fc_b.reshape(1, 1), (1, 16))], axis=0)
    out = _sc_call(u2, i2, user_table.T, item_table.T, params)
    return out.reshape(_B, 1)


# speedup vs baseline: 23.4042x; 1.0908x over previous
import jax
import jax.numpy as jnp
from jax import lax
from jax.experimental import pallas as pl
from jax.experimental.pallas import tpu as pltpu
from jax.experimental.pallas import tpu_sc as plsc

_B = 16384
_D = 32
_NC = 2
_NS = 16
_NW = _NC * _NS
_BPW = _B // _NW
_CW = 4
_NCH = _BPW // _CW


def _sc_body(user_ref, item_ref, ut_ref, it_ref, par_ref, out_ref,
             idx_u, idx_i, fb_u, fb_i, out_v, w_b,
             sem_u0, sem_u1, sem_u2, sem_i0, sem_i1, sem_i2):
    wid = lax.axis_index("s") * _NC + lax.axis_index("c")
    pltpu.sync_copy(user_ref.at[wid], idx_u.at[pl.ds(0, _BPW)])
    pltpu.sync_copy(item_ref.at[wid], idx_i.at[pl.ds(0, _BPW)])
    pltpu.sync_copy(par_ref, w_b)

    def start_chunk(c, bsel, sem_u, sem_i):
        iu = idx_u[pl.ds(c * _CW, 16)]
        ii = idx_i[pl.ds(c * _CW, 16)]
        for r in range(_CW):
            su = pl.multiple_of((iu[r] // 128) * 128, 128)
            si = pl.multiple_of((ii[r] // 128) * 128, 128)
            pltpu.async_copy(ut_ref.at[:, pl.ds(su, 128)], fb_u.at[bsel, r], sem_u)
            pltpu.async_copy(it_ref.at[:, pl.ds(si, 128)], fb_i.at[bsel, r], sem_i)

    def start_chunk_guarded(c, bsel, sem_u, sem_i):
        @pl.when(c < _NCH)
        def _():
            start_chunk(c, bsel, sem_u, sem_i)

    def drain(bsel, sem_u, sem_i):
        pltpu.make_async_copy(ut_ref.at[:, pl.ds(0, _CW * 128)],
                              fb_u.at[bsel], sem_u).wait()
        pltpu.make_async_copy(it_ref.at[:, pl.ds(0, _CW * 128)],
                              fb_i.at[bsel], sem_i).wait()

    wu0 = w_b[0]
    wu1 = w_b[1]
    wi0 = w_b[2]
    wi1 = w_b[3]
    bias = w_b[4]
    lane = jnp.arange(16, dtype=jnp.int32)
    ci = jnp.arange(16, dtype=jnp.int32)

    def compute_chunk(c, bsel, acc):
        acc = jnp.where((c % 4) == 0, bias, acc)
        iu = idx_u[pl.ds(c * _CW, 16)]
        ii = idx_i[pl.ds(c * _CW, 16)]
        for r in range(_CW):
            rlu = jnp.full((16,), iu[r] & 127, jnp.int32)
            rli = jnp.full((16,), ii[r] & 127, jnp.int32)
            gu0 = plsc.load_gather(fb_u.at[bsel, r], [ci, rlu])
            gu1 = plsc.load_gather(fb_u.at[bsel, r], [ci + 16, rlu])
            gi0 = plsc.load_gather(fb_i.at[bsel, r], [ci, rli])
            gi1 = plsc.load_gather(fb_i.at[bsel, r], [ci + 16, rli])
            t = gu0 * wu0 + gu1 * wu1 + gi0 * wi0 + gi1 * wi1
            acc = jnp.where(lane == (c % 4) * _CW + r, acc + jnp.sum(t), acc)

        @pl.when(((c % 4) == 3) & (c < _NCH))
        def _():
            out_v[pl.ds((c // 4) * 16, 16)] = 1.0 / (1.0 + jnp.exp(-acc))
        return acc

    sems = ((sem_u0, sem_i0), (sem_u1, sem_i1), (sem_u2, sem_i2))
    start_chunk(0, 0, *sems[0])
    start_chunk(1, 1, *sems[1])

    def step(k, acc):
        for p in range(3):
            c = 3 * k + p
            nxt = (p + 2) % 3
            start_chunk_guarded(c + 2, nxt, *sems[nxt])

            @pl.when(c < _NCH)
            def _():
                drain(p, *sems[p])
            acc2 = compute_chunk(c, p, acc)
            acc = jnp.where(c < _NCH, acc2, acc)
        return acc

    lax.fori_loop(0, (_NCH + 2) // 3, step, jnp.zeros((16,), jnp.float32))
    pltpu.sync_copy(out_v, out_ref.at[pl.ds(wid * _BPW, _BPW)])


_sc_call = pl.kernel(
    _sc_body,
    out_type=jax.ShapeDtypeStruct((_B,), jnp.float32),
    mesh=plsc.VectorSubcoreMesh(core_axis_name="c", subcore_axis_name="s"),
    scratch_types=[
        pltpu.VMEM((_BPW + 32,), jnp.int32),
        pltpu.VMEM((_BPW + 32,), jnp.int32),
        pltpu.VMEM((3, _CW, _D, 128), jnp.float32),
        pltpu.VMEM((3, _CW, _D, 128), jnp.float32),
        pltpu.VMEM((_BPW,), jnp.float32),
        pltpu.VMEM((5, 16), jnp.float32),
        pltpu.SemaphoreType.DMA,
        pltpu.SemaphoreType.DMA,
        pltpu.SemaphoreType.DMA,
        pltpu.SemaphoreType.DMA,
        pltpu.SemaphoreType.DMA,
        pltpu.SemaphoreType.DMA,
    ],
    compiler_params=pltpu.CompilerParams(
        needs_layout_passes=False, use_tc_tiling_on_sc=True),
)


def kernel(user, item, user_table, item_table, fc_w, fc_b):
    u2 = user.reshape(_NW, _BPW)
    i2 = item.reshape(_NW, _BPW)
    w = fc_w.reshape(4, 16)
    params = jnp.concatenate(
        [w, jnp.broadcast_to(fc_b.reshape(1, 1), (1, 16))], axis=0)
    out = _sc_call(u2, i2, user_table.T, item_table.T, params)
    return out.reshape(_B, 1)
